# baseline (speedup 1.0000x reference)
import jax
import jax.numpy as jnp
from jax import lax
from jax.experimental import pallas as pl
from jax.experimental.pallas import tpu as pltpu

G = 4
N = 512
NOISE = 128
CE = 64
HID = 512
FEAT = 256
NC = 10
M = N * (N - 1) // 2
K = int(round(0.05 * M))


def _dense_body(z_ref, ce_ref, w1_ref, b1_ref, w2_ref, b2_ref,
                we_ref, be_ref, h_ref, x_ref, gram_ref):
    z2 = z_ref[0]
    ce = ce_ref[0]
    inp = jnp.concatenate([z2, jnp.broadcast_to(ce, (N, CE))], axis=1)
    h1 = jnp.maximum(
        jnp.dot(inp.astype(jnp.bfloat16), w1_ref[...].astype(jnp.bfloat16),
                preferred_element_type=jnp.float32) + b1_ref[...], 0.0)
    x = (jnp.dot(h1.astype(jnp.bfloat16), w2_ref[...].astype(jnp.bfloat16),
                 preferred_element_type=jnp.float32) + b2_ref[...])
    x_ref[0] = x
    h = jnp.maximum(
        jnp.dot(x.astype(jnp.bfloat16), we_ref[...].astype(jnp.bfloat16),
                preferred_element_type=jnp.float32) + be_ref[...], 0.0)
    h_ref[0] = h
    hb = h.astype(jnp.bfloat16)
    gram_ref[0] = lax.dot_general(hb, hb, (((1,), (1,)), ((), ())),
                                  preferred_element_type=jnp.float32)


def kernel(z, class_labels, emb_table, W1, b1, W2, b2, We, be, threshold):
    ce = jnp.take(emb_table, class_labels, axis=0).reshape(G, 1, CE)

    h, x, gram = pl.pallas_call(
        _dense_body,
        grid=(G,),
        in_specs=[
            pl.BlockSpec((1, N, NOISE), lambda g: (g, 0, 0)),
            pl.BlockSpec((1, 1, CE), lambda g: (g, 0, 0)),
            pl.BlockSpec((NOISE + CE, HID), lambda g: (0, 0)),
            pl.BlockSpec((1, HID), lambda g: (0, 0)),
            pl.BlockSpec((HID, FEAT), lambda g: (0, 0)),
            pl.BlockSpec((1, FEAT), lambda g: (0, 0)),
            pl.BlockSpec((FEAT, HID), lambda g: (0, 0)),
            pl.BlockSpec((1, HID), lambda g: (0, 0)),
        ],
        out_specs=[
            pl.BlockSpec((1, N, HID), lambda g: (g, 0, 0)),
            pl.BlockSpec((1, N, FEAT), lambda g: (g, 0, 0)),
            pl.BlockSpec((1, N, N), lambda g: (g, 0, 0)),
        ],
        out_shape=[
            jax.ShapeDtypeStruct((G, N, HID), jnp.float32),
            jax.ShapeDtypeStruct((G, N, FEAT), jnp.float32),
            jax.ShapeDtypeStruct((G, N, N), jnp.float32),
        ],
        compiler_params=pltpu.CompilerParams(
            dimension_semantics=("arbitrary",)),
    )(z, ce, W1, b1.reshape(1, HID), W2, b2.reshape(1, FEAT),
      We, be.reshape(1, HID))

    sq = jnp.sum(h * h, axis=-1)
    d2 = sq[:, :, None] + sq[:, None, :] - 2.0 * gram
    d = jnp.sqrt(jnp.maximum(d2, 1e-12))
    p = jax.nn.sigmoid((-d + threshold) / 1.0)
    row_i = lax.broadcasted_iota(jnp.int32, (N, N), 0)
    col_j = lax.broadcasted_iota(jnp.int32, (N, N), 1)
    p = jnp.where(col_j > row_i, p, -1.0)

    topv, topi = lax.top_k(p.reshape(G, N * N), K)
    u = topi >> 9
    v = topi & (N - 1)
    offsets = (jnp.arange(G, dtype=u.dtype) * N)[:, None]
    u = (u + offsets).reshape(-1)
    v = (v + offsets).reshape(-1)
    edge_index = jnp.concatenate(
        [jnp.stack([u, v], axis=0), jnp.stack([v, u], axis=0)], axis=1)
    x_flat = x.reshape(G * N, FEAT)
    return (x_flat, topv, edge_index)


# speedup vs baseline: 10.1367x; 10.1367x over previous
import functools

import jax
import jax.numpy as jnp
from jax import lax
from jax.experimental import pallas as pl
from jax.experimental.pallas import tpu as pltpu
from jax.experimental.pallas import tpu_sc as plsc

G = 4
N = 512
NOISE = 128
CE = 64
HID = 512
FEAT = 256
NC = 10
M = N * (N - 1) // 2
K = int(round(0.05 * M))
NN = N * N
NW = 32
CHUNK = NN // NW
CAP = 8192
PAD_KEY = -(2 ** 31)


def _dense_body(z_ref, ce_ref, w1_ref, b1_ref, w2_ref, b2_ref,
                we_ref, be_ref, h_ref, x_ref, gram_ref):
    z2 = z_ref[0]
    ce = ce_ref[0]
    inp = jnp.concatenate([z2, jnp.broadcast_to(ce, (N, CE))], axis=1)
    h1 = jnp.maximum(
        jnp.dot(inp.astype(jnp.bfloat16), w1_ref[...].astype(jnp.bfloat16),
                preferred_element_type=jnp.float32) + b1_ref[...], 0.0)
    x = (jnp.dot(h1.astype(jnp.bfloat16), w2_ref[...].astype(jnp.bfloat16),
                 preferred_element_type=jnp.float32) + b2_ref[...])
    x_ref[0] = x
    h = jnp.maximum(
        jnp.dot(x.astype(jnp.bfloat16), we_ref[...].astype(jnp.bfloat16),
                preferred_element_type=jnp.float32) + be_ref[...], 0.0)
    h_ref[0] = h
    hb = h.astype(jnp.bfloat16)
    gram_ref[0] = lax.dot_general(hb, hb, (((1,), (1,)), ((), ())),
                                  preferred_element_type=jnp.float32)


def _kth_body(keys_ref, t_ref):
    keys = keys_ref[0]

    def step(_, lohi):
        lo, hi = lohi
        mid = (lo + hi) >> 1
        cnt = jnp.sum((keys >= mid).astype(jnp.int32))
        big = cnt >= K
        return (jnp.where(big, mid, lo), jnp.where(big, hi, mid))

    lo, hi = lax.fori_loop(0, 30, step, (jnp.int32(0), jnp.int32(0x40000000)))
    t_ref[0] = lo.reshape(1, 1)


def _compact_body(keys_hbm, t_hbm, outk_hbm, outi_hbm,
                  keys_v, bufk_v, bufi_v, t_v):
    wid = lax.axis_index("s") * 2 + lax.axis_index("c")
    lane = jnp.arange(16, dtype=jnp.int32)
    for g in range(G):
        pltpu.sync_copy(keys_hbm.at[pl.ds(g * NN + wid * CHUNK, CHUNK)],
                        keys_v)
        pltpu.sync_copy(t_hbm.at[pl.ds(g * 16, 16)], t_v)
        t16 = t_v[...]
        base = wid * CHUNK

        def step(i, runv):
            k16 = jnp.maximum(keys_v[pl.ds(i * 16, 16)], -1)
            sel = 1 - lax.shift_right_logical(k16 - t_v[...], 31)
            csum = plsc.cumsum(sel)
            pos = sel * (csum + runv - 1) + (1 - sel) * (jnp.int32(CHUNK) + lane)
            plsc.store_scatter(bufk_v, [pos], k16)
            idx16 = (base + i * 16) + lane
            plsc.store_scatter(bufi_v, [pos], idx16)
            total = plsc.cummax(lax.rev(csum, dimensions=(0,)))
            return runv + total

        lax.fori_loop(0, CHUNK // 16, step, jnp.zeros((16,), jnp.int32))
        pltpu.sync_copy(bufk_v.at[pl.ds(0, CHUNK)],
                        outk_hbm.at[pl.ds(g * NN + wid * CHUNK, CHUNK)])
        pltpu.sync_copy(bufi_v.at[pl.ds(0, CHUNK)],
                        outi_hbm.at[pl.ds(g * NN + wid * CHUNK, CHUNK)])


def _sort_body(pk_ref, pi_ref, keys_ref, t_ref, sk_ref, si_ref):
    t = t_ref[0, 0, 0]
    row = lax.broadcasted_iota(jnp.int32, (64, 128), 0)
    col = lax.broadcasted_iota(jnp.int32, (64, 128), 1)
    flat = row * 128 + col
    acck = jnp.zeros((64, 128), jnp.int32)
    acci = jnp.zeros((64, 128), jnp.int32)
    base = jnp.int32(0)
    for w in range(NW):
        cw = jnp.sum((keys_ref[0, pl.ds(w * 16, 16), :] >= t)
                     .astype(jnp.int32))
        dk = pk_ref[0, pl.ds(w * 64, 64), :]
        di = pi_ref[0, pl.ds(w * 64, 64), :]
        q = base >> 7
        r = base & 127
        m = (flat >= base) & (flat < base + cw)

        def flat_roll(a):
            b = pltpu.roll(a, r, 1)
            return jnp.where(col >= r, pltpu.roll(b, q, 0),
                             pltpu.roll(b, q + 1, 0))

        acck = jnp.where(m, flat_roll(dk), acck)
        acci = jnp.where(m, flat_roll(di), acci)
        base = base + cw
    live = flat < base
    ka = jnp.where(live, acck, PAD_KEY)
    ia = jnp.where(live, acci, jnp.int32(NN))
    k = 2
    while k <= CAP:
        j = k // 2
        while j >= 1:
            if j < 128:
                bit = (col & j) != 0
                kb = jnp.where(bit, jnp.roll(ka, j, axis=1),
                               jnp.roll(ka, -j, axis=1))
                ib = jnp.where(bit, jnp.roll(ia, j, axis=1),
                               jnp.roll(ia, -j, axis=1))
            else:
                jr = j >> 7
                bit = (row & jr) != 0
                kb = jnp.where(bit, jnp.roll(ka, jr, axis=0),
                               jnp.roll(ka, -jr, axis=0))
                ib = jnp.where(bit, jnp.roll(ia, jr, axis=0),
                               jnp.roll(ia, -jr, axis=0))
            greater = (ka > kb) | ((ka == kb) & (ia < ib))
            dirbit = ((flat & k) == 0) ^ bit
            keep = greater == dirbit
            ka = jnp.where(keep, ka, kb)
            ia = jnp.where(keep, ia, ib)
            j //= 2
        k *= 2
    sk_ref[0] = ka
    si_ref[0] = ia


def kernel(z, class_labels, emb_table, W1, b1, W2, b2, We, be, threshold):
    ce = jnp.take(emb_table, class_labels, axis=0).reshape(G, 1, CE)

    h, x, gram = pl.pallas_call(
        _dense_body,
        grid=(G,),
        in_specs=[
            pl.BlockSpec((1, N, NOISE), lambda g: (g, 0, 0)),
            pl.BlockSpec((1, 1, CE), lambda g: (g, 0, 0)),
            pl.BlockSpec((NOISE + CE, HID), lambda g: (0, 0)),
            pl.BlockSpec((1, HID), lambda g: (0, 0)),
            pl.BlockSpec((HID, FEAT), lambda g: (0, 0)),
            pl.BlockSpec((1, FEAT), lambda g: (0, 0)),
            pl.BlockSpec((FEAT, HID), lambda g: (0, 0)),
            pl.BlockSpec((1, HID), lambda g: (0, 0)),
        ],
        out_specs=[
            pl.BlockSpec((1, N, HID), lambda g: (g, 0, 0)),
            pl.BlockSpec((1, N, FEAT), lambda g: (g, 0, 0)),
            pl.BlockSpec((1, N, N), lambda g: (g, 0, 0)),
        ],
        out_shape=[
            jax.ShapeDtypeStruct((G, N, HID), jnp.float32),
            jax.ShapeDtypeStruct((G, N, FEAT), jnp.float32),
            jax.ShapeDtypeStruct((G, N, N), jnp.float32),
        ],
        compiler_params=pltpu.CompilerParams(
            dimension_semantics=("arbitrary",)),
    )(z, ce, W1, b1.reshape(1, HID), W2, b2.reshape(1, FEAT),
      We, be.reshape(1, HID))

    sq = jnp.sum(h * h, axis=-1)
    d2 = sq[:, :, None] + sq[:, None, :] - 2.0 * gram
    d = jnp.sqrt(jnp.maximum(d2, 1e-12))
    p = jax.nn.sigmoid((-d + threshold) / 1.0)
    row_i = lax.broadcasted_iota(jnp.int32, (N, N), 0)
    col_j = lax.broadcasted_iota(jnp.int32, (N, N), 1)
    p = jnp.where(col_j > row_i, p, -1.0)
    keys = lax.bitcast_convert_type(p, jnp.int32)

    t = pl.pallas_call(
        _kth_body,
        grid=(G,),
        in_specs=[pl.BlockSpec((1, N, N), lambda g: (g, 0, 0))],
        out_specs=pl.BlockSpec((1, 1, 1), lambda g: (g, 0, 0)),
        out_shape=jax.ShapeDtypeStruct((G, 1, 1), jnp.int32),
        compiler_params=pltpu.CompilerParams(
            dimension_semantics=("arbitrary",)),
    )(keys)

    t_b = jnp.broadcast_to(t.reshape(G, 1), (G, 16)).reshape(-1)

    mesh = plsc.VectorSubcoreMesh(core_axis_name="c", subcore_axis_name="s")
    compact = functools.partial(
        pl.kernel, mesh=mesh,
        out_type=[
            jax.ShapeDtypeStruct((G * NN,), jnp.int32),
            jax.ShapeDtypeStruct((G * NN,), jnp.int32),
        ],
        scratch_types=[
            pltpu.VMEM((CHUNK,), jnp.int32),
            pltpu.VMEM((CHUNK + 16,), jnp.int32),
            pltpu.VMEM((CHUNK + 16,), jnp.int32),
            pltpu.VMEM((16,), jnp.int32),
        ],
        compiler_params=pltpu.CompilerParams(needs_layout_passes=False),
    )(_compact_body)
    pk_, pi_ = compact(keys.reshape(-1), t_b)

    sk, si = pl.pallas_call(
        _sort_body,
        grid=(G,),
        in_specs=[
            pl.BlockSpec((1, NN // 128, 128), lambda g: (g, 0, 0)),
            pl.BlockSpec((1, NN // 128, 128), lambda g: (g, 0, 0)),
            pl.BlockSpec((1, N, N), lambda g: (g, 0, 0)),
            pl.BlockSpec((1, 1, 1), lambda g: (g, 0, 0)),
        ],
        out_specs=[
            pl.BlockSpec((1, 64, 128), lambda g: (g, 0, 0)),
            pl.BlockSpec((1, 64, 128), lambda g: (g, 0, 0)),
        ],
        out_shape=[
            jax.ShapeDtypeStruct((G, 64, 128), jnp.int32),
            jax.ShapeDtypeStruct((G, 64, 128), jnp.int32),
        ],
        compiler_params=pltpu.CompilerParams(
            dimension_semantics=("arbitrary",)),
    )(pk_.reshape(G, NN // 128, 128), pi_.reshape(G, NN // 128, 128), keys, t)

    sk = sk.reshape(G, CAP)
    si = si.reshape(G, CAP)
    topv = lax.bitcast_convert_type(sk[:, :K], jnp.float32)
    topi = si[:, :K]
    u = topi >> 9
    v = topi & (N - 1)
    offsets = (jnp.arange(G, dtype=u.dtype) * N)[:, None]
    u = (u + offsets).reshape(-1)
    v = (v + offsets).reshape(-1)
    edge_index = jnp.concatenate(
        [jnp.stack([u, v], axis=0), jnp.stack([v, u], axis=0)], axis=1)
    x_flat = x.reshape(G * N, FEAT)
    return (x_flat, topv, edge_index)
